# baseline (speedup 1.0000x reference)
import functools

import jax
import jax.numpy as jnp
import numpy as np
from jax import lax
from jax.experimental import pallas as pl
from jax.experimental.pallas import tpu as pltpu
from jax.experimental.pallas import tpu_sc as plsc

N = 10000
E = 320000
D = 128
NC = 2
NS = 16
NW = NC * NS
EP = 327680
PER_W = EP // NW
CB = 128
CH = PER_W // CB
NP = 10240
NPS = NP // NS

_f32 = jnp.float32


def _proj_body(x_ref, ws_ref, wd_ref, fs_ref, fd_ref):
    x = x_ref[...]
    fs_ref[...] = jnp.dot(x, ws_ref[...], preferred_element_type=_f32)
    fd_ref[...] = jnp.dot(x, wd_ref[...], preferred_element_type=_f32)


def _project(x, Wsrc, Wdst):
    Bn = 400
    grid = (N // Bn,)
    return pl.pallas_call(
        _proj_body,
        grid=grid,
        in_specs=[
            pl.BlockSpec((Bn, D), lambda i: (i, 0)),
            pl.BlockSpec((D, D), lambda i: (0, 0)),
            pl.BlockSpec((D, D), lambda i: (0, 0)),
        ],
        out_specs=[
            pl.BlockSpec((Bn, D), lambda i: (i, 0)),
            pl.BlockSpec((Bn, D), lambda i: (i, 0)),
        ],
        out_shape=[
            jax.ShapeDtypeStruct((N, D), _f32),
            jax.ShapeDtypeStruct((N, D), _f32),
        ],
    )(x, Wsrc, Wdst)


def _edge_body(rs_ref, rd_ref, a_ref, rm_ref, msg_ref, ex_ref, *, block):
    rs = rs_ref[...]
    t = rs + rd_ref[...]
    t = jnp.maximum(t, 0.2 * t)
    logits = jnp.dot(t, a_ref[...], preferred_element_type=_f32)
    exr = jnp.dot(jnp.exp(logits), rm_ref[...], preferred_element_type=_f32)
    rowid = (jax.lax.broadcasted_iota(jnp.int32, (block, 1), 0)
             + pl.program_id(0) * block)
    exr = jnp.where(rowid < E, exr, 0.0)
    ex_ref[...] = exr
    msg_ref[...] = rs * exr


def _edge_math(rows_s, rows_d, A, Rm):
    Be = 2048
    grid = (EP // Be,)
    return pl.pallas_call(
        functools.partial(_edge_body, block=Be),
        grid=grid,
        in_specs=[
            pl.BlockSpec((Be, D), lambda i: (i, 0)),
            pl.BlockSpec((Be, D), lambda i: (i, 0)),
            pl.BlockSpec((D, 16), lambda i: (0, 0)),
            pl.BlockSpec((16, D), lambda i: (0, 0)),
        ],
        out_specs=[
            pl.BlockSpec((Be, D), lambda i: (i, 0)),
            pl.BlockSpec((Be, D), lambda i: (i, 0)),
        ],
        out_shape=[
            jax.ShapeDtypeStruct((EP, D), _f32),
            jax.ShapeDtypeStruct((EP, D), _f32),
        ],
    )(rows_s, rows_d, A, Rm)


def _post_body(u0_ref, u1_ref, d0_ref, d1_ref, bias_ref, g_ref, b_ref,
               res_ref, y_ref, *, residual):
    u = u0_ref[...] + u1_ref[...]
    den = d0_ref[...] + d1_ref[...] + 1e-9
    o = u / den + bias_ref[...]
    o = jnp.where(o > 0, o, jnp.exp(jnp.minimum(o, 0.0)) - 1.0)
    if residual:
        o = o + res_ref[...]
    mu = jnp.mean(o, axis=-1, keepdims=True)
    var = jnp.mean((o - mu) ** 2, axis=-1, keepdims=True)
    y_ref[...] = (o - mu) / jnp.sqrt(var + 1e-5) * g_ref[...] + b_ref[...]


def _postprocess(u0, u1, d0, d1, bias, g, b, res, residual):
    Bn = 400
    grid = (N // Bn,)
    return pl.pallas_call(
        functools.partial(_post_body, residual=residual),
        grid=grid,
        in_specs=[
            pl.BlockSpec((Bn, D), lambda i: (i, 0)),
            pl.BlockSpec((Bn, D), lambda i: (i, 0)),
            pl.BlockSpec((Bn, D), lambda i: (i, 0)),
            pl.BlockSpec((Bn, D), lambda i: (i, 0)),
            pl.BlockSpec((1, D), lambda i: (0, 0)),
            pl.BlockSpec((1, D), lambda i: (0, 0)),
            pl.BlockSpec((1, D), lambda i: (0, 0)),
            pl.BlockSpec((Bn, D), lambda i: (i, 0)),
        ],
        out_specs=pl.BlockSpec((Bn, D), lambda i: (i, 0)),
        out_shape=jax.ShapeDtypeStruct((N, D), _f32),
    )(u0, u1, d0, d1, bias, g, b, res)


_MESH = plsc.VectorSubcoreMesh(core_axis_name="c", subcore_axis_name="s",
                               num_cores=NC, num_subcores=NS)


@functools.partial(
    pl.kernel,
    out_type=[
        jax.ShapeDtypeStruct((EP, D), _f32),
        jax.ShapeDtypeStruct((EP, D), _f32),
    ],
    mesh=_MESH,
    scratch_types=[
        pltpu.VMEM((CH, CB), jnp.int32),
        pltpu.VMEM((CH, CB), jnp.int32),
        pltpu.VMEM((CB, D), _f32),
        pltpu.VMEM((CB, D), _f32),
        pltpu.VMEM((CB, D), _f32),
        pltpu.VMEM((CB, D), _f32),
        pltpu.SemaphoreType.DMA,
        pltpu.SemaphoreType.DMA,
        pltpu.SemaphoreType.DMA,
        pltpu.SemaphoreType.DMA,
    ],
)
def _sc_gather(fs_hbm, fd_hbm, srcr_hbm, dstr_hbm, rs_hbm, rd_hbm,
               sidx, didx, bs0, bd0, bs1, bd1, ss0, sd0, ss1, sd1):
    wid = lax.axis_index("s") * NC + lax.axis_index("c")
    pltpu.sync_copy(srcr_hbm.at[wid], sidx)
    pltpu.sync_copy(dstr_hbm.at[wid], didx)
    base = wid * PER_W

    def body(j, carry):
        cs0 = pltpu.async_copy(fs_hbm.at[sidx.at[j]], bs0, ss0)
        cd0 = pltpu.async_copy(fd_hbm.at[didx.at[j]], bd0, sd0)
        cs0.wait()
        cd0.wait()
        off0 = base + j * CB
        pltpu.sync_copy(bs0, rs_hbm.at[pl.ds(off0, CB)])
        pltpu.sync_copy(bd0, rd_hbm.at[pl.ds(off0, CB)])
        return carry

    lax.fori_loop(0, CH, body, 0)


@functools.partial(
    pl.kernel,
    out_type=jax.ShapeDtypeStruct((NC * NP, D), _f32),
    mesh=_MESH,
    scratch_types=[
        pltpu.VMEM((CB,), jnp.int32),
        pltpu.VMEM((CB,), jnp.int32),
        pltpu.VMEM((CB, D), _f32),
        pltpu.VMEM((CB, D), _f32),
        pltpu.VMEM_SHARED((NP, D), _f32),
        pltpu.SemaphoreType.DMA,
        pltpu.SemaphoreType.DMA,
        pltpu.SemaphoreType.DMA,
        pltpu.SemaphoreType.DMA,
    ],
)
def _sc_scatter_msg(msg_hbm, dst_hbm, u_hbm,
                    didx0, didx1, mbuf0, mbuf1, sh_out, si0, si1, sm0, sm1):
    c = lax.axis_index("c")
    s = lax.axis_index("s")
    wid = s * NC + c

    zv = jnp.zeros((16,), _f32)
    for r in range(CB):
        for kk in range(D // 16):
            mbuf0[r, pl.ds(kk * 16, 16)] = zv
    for i in range(NPS // CB):
        pltpu.sync_copy(mbuf0, sh_out.at[pl.ds(s * NPS + i * CB, CB)])
    plsc.subcore_barrier()
    base = wid * PER_W

    def body(j, carry):
        off0 = base + j * CB
        ci0 = pltpu.async_copy(dst_hbm.at[pl.ds(off0, CB)], didx0, si0)
        cm0 = pltpu.async_copy(msg_hbm.at[pl.ds(off0, CB)], mbuf0, sm0)
        ci0.wait()
        cm0.wait()
        pltpu.sync_copy(mbuf0, sh_out.at[didx0], add=True)
        return carry

    lax.fori_loop(0, CH, body, 0)
    plsc.subcore_barrier()
    for i in range(NPS // CB):
        rows = pl.ds(s * NPS + i * CB, CB)
        pltpu.sync_copy(sh_out.at[rows], mbuf0)
        pltpu.sync_copy(mbuf0, u_hbm.at[pl.ds(c * NP + s * NPS + i * CB, CB)])


def _head_mask(H, DH):
    M = np.zeros((D, 16), np.float32)
    for h in range(H):
        M[h * DH:(h + 1) * DH, h] = 1.0
    return M


_M1 = _head_mask(8, 16)
_M2 = _head_mask(1, 128)


def kernel(x, edge_index, W_src1, W_dst1, attn1, bias1, ln1_g, ln1_b,
           W_src2, W_dst2, attn2, bias2, ln2_g, ln2_b):
    ei = edge_index.astype(jnp.int32)
    srcp = jnp.pad(ei[0], (0, EP - E))
    dstp = jnp.pad(ei[1], (0, EP - E))
    srcr = srcp.reshape(NW, CH, CB)
    dstr = dstp.reshape(NW, CH, CB)


    def layer(h, Wsrc, Wdst, attn, bias, g, b, res, residual, M):
        fs, fd = _project(h, Wsrc, Wdst)
        A = attn.reshape(D)[:, None] * jnp.asarray(M)
        Rm = jnp.asarray(M.T)
        rows_s, rows_d = _sc_gather(fs, fd, srcr, dstr)
        msg, ex = _edge_math(rows_s, rows_d, A, Rm)
        u = _sc_scatter_msg(msg, dstp)
        dd = _sc_scatter_msg(ex, dstp)
        u0, u1 = u[:NP], u[NP:]
        d0, d1 = dd[:NP], dd[NP:]
        return _postprocess(u0, u1, d0, d1, bias.reshape(1, D),
                            g.reshape(1, D), b.reshape(1, D), res, residual)

    h = layer(x, W_src1, W_dst1, attn1, bias1, ln1_g, ln1_b, x, True, _M1)
    out = layer(h, W_src2, W_dst2, attn2, bias2, ln2_g, ln2_b, h, False, _M2)
    return out


# speedup vs baseline: 1.5562x; 1.5562x over previous
import functools

import jax
import jax.numpy as jnp
import numpy as np
from jax import lax
from jax.experimental import pallas as pl
from jax.experimental.pallas import tpu as pltpu
from jax.experimental.pallas import tpu_sc as plsc

N = 10000
E = 320000
D = 128
NC = 2
NS = 16
NW = NC * NS
EP = E
PER_W = EP // NW
CB = 80
CH = PER_W // CB
NP = 10240
NPS = NP // NS

_f32 = jnp.float32


def _proj_body(x_ref, ws_ref, wd_ref, fs_ref, fd_ref):
    x = x_ref[...]
    fs_ref[...] = jnp.dot(x, ws_ref[...], preferred_element_type=_f32)
    fd_ref[...] = jnp.dot(x, wd_ref[...], preferred_element_type=_f32)


def _project(x, Wsrc, Wdst):
    Bn = 400
    grid = (N // Bn,)
    return pl.pallas_call(
        _proj_body,
        grid=grid,
        in_specs=[
            pl.BlockSpec((Bn, D), lambda i: (i, 0)),
            pl.BlockSpec((D, D), lambda i: (0, 0)),
            pl.BlockSpec((D, D), lambda i: (0, 0)),
        ],
        out_specs=[
            pl.BlockSpec((Bn, D), lambda i: (i, 0)),
            pl.BlockSpec((Bn, D), lambda i: (i, 0)),
        ],
        out_shape=[
            jax.ShapeDtypeStruct((N, D), _f32),
            jax.ShapeDtypeStruct((N, D), _f32),
        ],
    )(x, Wsrc, Wdst)


def _edge_body(rs_ref, rd_ref, a_ref, rm_ref, msg_ref, ex_ref, *, block):
    rs = rs_ref[...]
    t = rs + rd_ref[...]
    t = jnp.maximum(t, 0.2 * t)
    logits = jnp.dot(t, a_ref[...], preferred_element_type=_f32)
    exr = jnp.dot(jnp.exp(logits), rm_ref[...], preferred_element_type=_f32)
    rowid = (jax.lax.broadcasted_iota(jnp.int32, (block, 1), 0)
             + pl.program_id(0) * block)
    exr = jnp.where(rowid < E, exr, 0.0)
    ex_ref[...] = exr
    msg_ref[...] = rs * exr


def _edge_math(rows_s, rows_d, A, Rm):
    Be = 2000
    grid = (EP // Be,)
    return pl.pallas_call(
        functools.partial(_edge_body, block=Be),
        grid=grid,
        in_specs=[
            pl.BlockSpec((Be, D), lambda i: (i, 0)),
            pl.BlockSpec((Be, D), lambda i: (i, 0)),
            pl.BlockSpec((D, 16), lambda i: (0, 0)),
            pl.BlockSpec((16, D), lambda i: (0, 0)),
        ],
        out_specs=[
            pl.BlockSpec((Be, D), lambda i: (i, 0)),
            pl.BlockSpec((Be, D), lambda i: (i, 0)),
        ],
        out_shape=[
            jax.ShapeDtypeStruct((EP, D), _f32),
            jax.ShapeDtypeStruct((EP, D), _f32),
        ],
    )(rows_s, rows_d, A, Rm)


def _post_body(u0_ref, u1_ref, d0_ref, d1_ref, bias_ref, g_ref, b_ref,
               res_ref, y_ref, *, residual):
    u = u0_ref[...] + u1_ref[...]
    den = d0_ref[...] + d1_ref[...] + 1e-9
    o = u / den + bias_ref[...]
    o = jnp.where(o > 0, o, jnp.exp(jnp.minimum(o, 0.0)) - 1.0)
    if residual:
        o = o + res_ref[...]
    mu = jnp.mean(o, axis=-1, keepdims=True)
    var = jnp.mean((o - mu) ** 2, axis=-1, keepdims=True)
    y_ref[...] = (o - mu) / jnp.sqrt(var + 1e-5) * g_ref[...] + b_ref[...]


def _postprocess(u0, u1, d0, d1, bias, g, b, res, residual):
    Bn = 400
    grid = (N // Bn,)
    return pl.pallas_call(
        functools.partial(_post_body, residual=residual),
        grid=grid,
        in_specs=[
            pl.BlockSpec((Bn, D), lambda i: (i, 0)),
            pl.BlockSpec((Bn, D), lambda i: (i, 0)),
            pl.BlockSpec((Bn, D), lambda i: (i, 0)),
            pl.BlockSpec((Bn, D), lambda i: (i, 0)),
            pl.BlockSpec((1, D), lambda i: (0, 0)),
            pl.BlockSpec((1, D), lambda i: (0, 0)),
            pl.BlockSpec((1, D), lambda i: (0, 0)),
            pl.BlockSpec((Bn, D), lambda i: (i, 0)),
        ],
        out_specs=pl.BlockSpec((Bn, D), lambda i: (i, 0)),
        out_shape=jax.ShapeDtypeStruct((N, D), _f32),
    )(u0, u1, d0, d1, bias, g, b, res)


_MESH = plsc.VectorSubcoreMesh(core_axis_name="c", subcore_axis_name="s",
                               num_cores=NC, num_subcores=NS)


@functools.partial(
    pl.kernel,
    out_type=[
        jax.ShapeDtypeStruct((EP, D), _f32),
        jax.ShapeDtypeStruct((EP, D), _f32),
    ],
    mesh=_MESH,
    scratch_types=[
        pltpu.VMEM((CH, CB), jnp.int32),
        pltpu.VMEM((CH, CB), jnp.int32),
        pltpu.VMEM((CB, D), _f32),
        pltpu.VMEM((CB, D), _f32),
        pltpu.VMEM((CB, D), _f32),
        pltpu.VMEM((CB, D), _f32),
        pltpu.SemaphoreType.DMA,
        pltpu.SemaphoreType.DMA,
        pltpu.SemaphoreType.DMA,
        pltpu.SemaphoreType.DMA,
    ],
)
def _sc_gather(fs_hbm, fd_hbm, srcr_hbm, dstr_hbm, rs_hbm, rd_hbm,
               sidx, didx, bs0, bd0, bs1, bd1, ss0, sd0, ss1, sd1):
    wid = lax.axis_index("s") * NC + lax.axis_index("c")
    pltpu.sync_copy(srcr_hbm.at[wid], sidx)
    pltpu.sync_copy(dstr_hbm.at[wid], didx)
    base = wid * PER_W

    def body(g, carry):
        j0 = g * 2
        j1 = g * 2 + 1
        cs0 = pltpu.async_copy(fs_hbm.at[sidx.at[j0]], bs0, ss0)
        cd0 = pltpu.async_copy(fd_hbm.at[didx.at[j0]], bd0, sd0)
        cs1 = pltpu.async_copy(fs_hbm.at[sidx.at[j1]], bs1, ss1)
        cd1 = pltpu.async_copy(fd_hbm.at[didx.at[j1]], bd1, sd1)
        cs0.wait()
        cd0.wait()
        off0 = base + j0 * CB
        pltpu.sync_copy(bs0, rs_hbm.at[pl.ds(off0, CB)])
        pltpu.sync_copy(bd0, rd_hbm.at[pl.ds(off0, CB)])
        cs1.wait()
        cd1.wait()
        off1 = base + j1 * CB
        pltpu.sync_copy(bs1, rs_hbm.at[pl.ds(off1, CB)])
        pltpu.sync_copy(bd1, rd_hbm.at[pl.ds(off1, CB)])
        return carry

    lax.fori_loop(0, CH // 2, body, 0)
    if CH % 2:
        j = CH - 1
        cs = pltpu.async_copy(fs_hbm.at[sidx.at[j]], bs0, ss0)
        cd = pltpu.async_copy(fd_hbm.at[didx.at[j]], bd0, sd0)
        cs.wait()
        cd.wait()
        off = base + j * CB
        pltpu.sync_copy(bs0, rs_hbm.at[pl.ds(off, CB)])
        pltpu.sync_copy(bd0, rd_hbm.at[pl.ds(off, CB)])


@functools.partial(
    pl.kernel,
    out_type=jax.ShapeDtypeStruct((NC * NP, D), _f32),
    mesh=_MESH,
    scratch_types=[
        pltpu.VMEM((CB,), jnp.int32),
        pltpu.VMEM((CB,), jnp.int32),
        pltpu.VMEM((CB, D), _f32),
        pltpu.VMEM((CB, D), _f32),
        pltpu.VMEM_SHARED((NP, D), _f32),
        pltpu.SemaphoreType.DMA,
        pltpu.SemaphoreType.DMA,
        pltpu.SemaphoreType.DMA,
        pltpu.SemaphoreType.DMA,
    ],
)
def _sc_scatter_msg(msg_hbm, dst_hbm, u_hbm,
                    didx0, didx1, mbuf0, mbuf1, sh_out, si0, si1, sm0, sm1):
    c = lax.axis_index("c")
    s = lax.axis_index("s")
    wid = s * NC + c

    zv = jnp.zeros((16,), _f32)
    for r in range(CB):
        for kk in range(D // 16):
            mbuf0[r, pl.ds(kk * 16, 16)] = zv
    for i in range(NPS // CB):
        pltpu.sync_copy(mbuf0, sh_out.at[pl.ds(s * NPS + i * CB, CB)])
    plsc.subcore_barrier()
    base = wid * PER_W

    def body(g, carry):
        off0 = base + (g * 2) * CB
        off1 = base + (g * 2 + 1) * CB
        ci0 = pltpu.async_copy(dst_hbm.at[pl.ds(off0, CB)], didx0, si0)
        cm0 = pltpu.async_copy(msg_hbm.at[pl.ds(off0, CB)], mbuf0, sm0)
        ci1 = pltpu.async_copy(dst_hbm.at[pl.ds(off1, CB)], didx1, si1)
        cm1 = pltpu.async_copy(msg_hbm.at[pl.ds(off1, CB)], mbuf1, sm1)
        ci0.wait()
        cm0.wait()
        pltpu.sync_copy(mbuf0, sh_out.at[didx0], add=True)
        ci1.wait()
        cm1.wait()
        pltpu.sync_copy(mbuf1, sh_out.at[didx1], add=True)
        return carry

    lax.fori_loop(0, CH // 2, body, 0)
    if CH % 2:
        off = base + (CH - 1) * CB
        ci = pltpu.async_copy(dst_hbm.at[pl.ds(off, CB)], didx0, si0)
        cm = pltpu.async_copy(msg_hbm.at[pl.ds(off, CB)], mbuf0, sm0)
        ci.wait()
        cm.wait()
        pltpu.sync_copy(mbuf0, sh_out.at[didx0], add=True)
    plsc.subcore_barrier()
    for i in range(NPS // CB):
        rows = pl.ds(s * NPS + i * CB, CB)
        pltpu.sync_copy(sh_out.at[rows], mbuf0)
        pltpu.sync_copy(mbuf0, u_hbm.at[pl.ds(c * NP + s * NPS + i * CB, CB)])


def _head_mask(H, DH):
    M = np.zeros((D, 16), np.float32)
    for h in range(H):
        M[h * DH:(h + 1) * DH, h] = 1.0
    return M


_M1 = _head_mask(8, 16)
_M2 = _head_mask(1, 128)


def kernel(x, edge_index, W_src1, W_dst1, attn1, bias1, ln1_g, ln1_b,
           W_src2, W_dst2, attn2, bias2, ln2_g, ln2_b):
    ei = edge_index.astype(jnp.int32)
    srcp = jnp.pad(ei[0], (0, EP - E))
    dstp = jnp.pad(ei[1], (0, EP - E))
    srcr = srcp.reshape(NW, CH, CB)
    dstr = dstp.reshape(NW, CH, CB)


    def layer(h, Wsrc, Wdst, attn, bias, g, b, res, residual, M):
        fs, fd = _project(h, Wsrc, Wdst)
        A = attn.reshape(D)[:, None] * jnp.asarray(M)
        Rm = jnp.asarray(M.T)
        rows_s, rows_d = _sc_gather(fs, fd, srcr, dstr)
        msg, ex = _edge_math(rows_s, rows_d, A, Rm)
        u = _sc_scatter_msg(msg, dstp)
        dd = _sc_scatter_msg(ex, dstp)
        u0, u1 = u[:NP], u[NP:]
        d0, d1 = dd[:NP], dd[NP:]
        return _postprocess(u0, u1, d0, d1, bias.reshape(1, D),
                            g.reshape(1, D), b.reshape(1, D), res, residual)

    h = layer(x, W_src1, W_dst1, attn1, bias1, ln1_g, ln1_b, x, True, _M1)
    out = layer(h, W_src2, W_dst2, attn2, bias2, ln2_g, ln2_b, h, False, _M2)
    return out


# speedup vs baseline: 1.6165x; 1.0387x over previous
import functools

import jax
import jax.numpy as jnp
import numpy as np
from jax import lax
from jax.experimental import pallas as pl
from jax.experimental.pallas import tpu as pltpu
from jax.experimental.pallas import tpu_sc as plsc

N = 10000
E = 320000
D = 128
NC = 2
NS = 16
NW = NC * NS
EP = E
PER_W = EP // NW
CB = 80
CH = PER_W // CB
NP = 10240
NPS = NP // NS

_f32 = jnp.float32


def _proj_body(x_ref, ws_ref, wd_ref, fs_ref, fd_ref):
    x = x_ref[...]
    fs_ref[...] = jnp.dot(x, ws_ref[...], preferred_element_type=_f32)
    fd_ref[...] = jnp.dot(x, wd_ref[...], preferred_element_type=_f32)


def _project(x, Wsrc, Wdst):
    Bn = 400
    grid = (N // Bn,)
    return pl.pallas_call(
        _proj_body,
        grid=grid,
        in_specs=[
            pl.BlockSpec((Bn, D), lambda i: (i, 0)),
            pl.BlockSpec((D, D), lambda i: (0, 0)),
            pl.BlockSpec((D, D), lambda i: (0, 0)),
        ],
        out_specs=[
            pl.BlockSpec((Bn, D), lambda i: (i, 0)),
            pl.BlockSpec((Bn, D), lambda i: (i, 0)),
        ],
        out_shape=[
            jax.ShapeDtypeStruct((N, D), _f32),
            jax.ShapeDtypeStruct((N, D), _f32),
        ],
    )(x, Wsrc, Wdst)


def _edge_body(rs_ref, rd_ref, a_ref, rm_ref, msg_ref, ex_ref, *, block):
    rs = rs_ref[...]
    t = rs + rd_ref[...]
    t = jnp.maximum(t, 0.2 * t)
    logits = jnp.dot(t, a_ref[...], preferred_element_type=_f32)
    exr = jnp.dot(jnp.exp(logits), rm_ref[...], preferred_element_type=_f32)
    rowid = (jax.lax.broadcasted_iota(jnp.int32, (block, 1), 0)
             + pl.program_id(0) * block)
    exr = jnp.where(rowid < E, exr, 0.0)
    ex_ref[...] = exr
    msg_ref[...] = rs * exr


def _edge_math(rows_s, rows_d, A, Rm):
    Be = 2000
    grid = (EP // Be,)
    return pl.pallas_call(
        functools.partial(_edge_body, block=Be),
        grid=grid,
        in_specs=[
            pl.BlockSpec((Be, D), lambda i: (i, 0)),
            pl.BlockSpec((Be, D), lambda i: (i, 0)),
            pl.BlockSpec((D, 16), lambda i: (0, 0)),
            pl.BlockSpec((16, D), lambda i: (0, 0)),
        ],
        out_specs=[
            pl.BlockSpec((Be, D), lambda i: (i, 0)),
            pl.BlockSpec((Be, D), lambda i: (i, 0)),
        ],
        out_shape=[
            jax.ShapeDtypeStruct((EP, D), _f32),
            jax.ShapeDtypeStruct((EP, D), _f32),
        ],
    )(rows_s, rows_d, A, Rm)


def _post_body(u0_ref, u1_ref, d0_ref, d1_ref, bias_ref, g_ref, b_ref,
               res_ref, y_ref, *, residual):
    u = u0_ref[...] + u1_ref[...]
    den = d0_ref[...] + d1_ref[...] + 1e-9
    o = u / den + bias_ref[...]
    o = jnp.where(o > 0, o, jnp.exp(jnp.minimum(o, 0.0)) - 1.0)
    if residual:
        o = o + res_ref[...]
    mu = jnp.mean(o, axis=-1, keepdims=True)
    var = jnp.mean((o - mu) ** 2, axis=-1, keepdims=True)
    y_ref[...] = (o - mu) / jnp.sqrt(var + 1e-5) * g_ref[...] + b_ref[...]


def _postprocess(u0, u1, d0, d1, bias, g, b, res, residual):
    Bn = 400
    grid = (N // Bn,)
    return pl.pallas_call(
        functools.partial(_post_body, residual=residual),
        grid=grid,
        in_specs=[
            pl.BlockSpec((Bn, D), lambda i: (i, 0)),
            pl.BlockSpec((Bn, D), lambda i: (i, 0)),
            pl.BlockSpec((Bn, D), lambda i: (i, 0)),
            pl.BlockSpec((Bn, D), lambda i: (i, 0)),
            pl.BlockSpec((1, D), lambda i: (0, 0)),
            pl.BlockSpec((1, D), lambda i: (0, 0)),
            pl.BlockSpec((1, D), lambda i: (0, 0)),
            pl.BlockSpec((Bn, D), lambda i: (i, 0)),
        ],
        out_specs=pl.BlockSpec((Bn, D), lambda i: (i, 0)),
        out_shape=jax.ShapeDtypeStruct((N, D), _f32),
    )(u0, u1, d0, d1, bias, g, b, res)


_MESH = plsc.VectorSubcoreMesh(core_axis_name="c", subcore_axis_name="s",
                               num_cores=NC, num_subcores=NS)


@functools.partial(
    pl.kernel,
    out_type=[
        jax.ShapeDtypeStruct((EP, D), _f32),
        jax.ShapeDtypeStruct((EP, D), _f32),
    ],
    mesh=_MESH,
    scratch_types=[
        pltpu.VMEM((CH, CB), jnp.int32),
        pltpu.VMEM((CH, CB), jnp.int32),
        pltpu.VMEM((CB, D), _f32),
        pltpu.VMEM((CB, D), _f32),
        pltpu.VMEM((CB, D), _f32),
        pltpu.VMEM((CB, D), _f32),
        pltpu.VMEM((CB, D), _f32),
        pltpu.VMEM((CB, D), _f32),
        pltpu.VMEM((CB, D), _f32),
        pltpu.VMEM((CB, D), _f32),
        pltpu.SemaphoreType.DMA,
        pltpu.SemaphoreType.DMA,
        pltpu.SemaphoreType.DMA,
        pltpu.SemaphoreType.DMA,
        pltpu.SemaphoreType.DMA,
        pltpu.SemaphoreType.DMA,
        pltpu.SemaphoreType.DMA,
        pltpu.SemaphoreType.DMA,
    ],
)
def _sc_gather(fs_hbm, fd_hbm, srcr_hbm, dstr_hbm, rs_hbm, rd_hbm,
               sidx, didx, bs0, bd0, bs1, bd1, bs2, bd2, bs3, bd3,
               ss0, sd0, ss1, sd1, ss2, sd2, ss3, sd3):
    wid = lax.axis_index("s") * NC + lax.axis_index("c")
    pltpu.sync_copy(srcr_hbm.at[wid], sidx)
    pltpu.sync_copy(dstr_hbm.at[wid], didx)
    base = wid * PER_W
    bufs = ((bs0, bd0, ss0, sd0), (bs1, bd1, ss1, sd1),
            (bs2, bd2, ss2, sd2), (bs3, bd3, ss3, sd3))

    def body(g, carry):
        cps = []
        for q, (bs, bd, ss, sd) in enumerate(bufs):
            j = g * 4 + q
            cps.append((pltpu.async_copy(fs_hbm.at[sidx.at[j]], bs, ss),
                        pltpu.async_copy(fd_hbm.at[didx.at[j]], bd, sd)))
        for q, (bs, bd, ss, sd) in enumerate(bufs):
            j = g * 4 + q
            cs, cd = cps[q]
            cs.wait()
            cd.wait()
            off = base + j * CB
            pltpu.sync_copy(bs, rs_hbm.at[pl.ds(off, CB)])
            pltpu.sync_copy(bd, rd_hbm.at[pl.ds(off, CB)])
        return carry

    lax.fori_loop(0, CH // 4, body, 0)
    for j in range(CH - CH % 4, CH):
        cs = pltpu.async_copy(fs_hbm.at[sidx.at[j]], bs0, ss0)
        cd = pltpu.async_copy(fd_hbm.at[didx.at[j]], bd0, sd0)
        cs.wait()
        cd.wait()
        off = base + j * CB
        pltpu.sync_copy(bs0, rs_hbm.at[pl.ds(off, CB)])
        pltpu.sync_copy(bd0, rd_hbm.at[pl.ds(off, CB)])


@functools.partial(
    pl.kernel,
    out_type=jax.ShapeDtypeStruct((NC * NP, D), _f32),
    mesh=_MESH,
    scratch_types=[
        pltpu.VMEM((CB,), jnp.int32),
        pltpu.VMEM((CB,), jnp.int32),
        pltpu.VMEM((CB, D), _f32),
        pltpu.VMEM((CB, D), _f32),
        pltpu.VMEM_SHARED((NP, D), _f32),
        pltpu.SemaphoreType.DMA,
        pltpu.SemaphoreType.DMA,
        pltpu.SemaphoreType.DMA,
        pltpu.SemaphoreType.DMA,
    ],
)
def _sc_scatter_msg(msg_hbm, dst_hbm, u_hbm,
                    didx0, didx1, mbuf0, mbuf1, sh_out, si0, si1, sm0, sm1):
    c = lax.axis_index("c")
    s = lax.axis_index("s")
    wid = s * NC + c

    zv = jnp.zeros((16,), _f32)
    for r in range(CB):
        for kk in range(D // 16):
            mbuf0[r, pl.ds(kk * 16, 16)] = zv
    for i in range(NPS // CB):
        pltpu.sync_copy(mbuf0, sh_out.at[pl.ds(s * NPS + i * CB, CB)])
    plsc.subcore_barrier()
    base = wid * PER_W

    def body(g, carry):
        off0 = base + (g * 2) * CB
        off1 = base + (g * 2 + 1) * CB
        ci0 = pltpu.async_copy(dst_hbm.at[pl.ds(off0, CB)], didx0, si0)
        cm0 = pltpu.async_copy(msg_hbm.at[pl.ds(off0, CB)], mbuf0, sm0)
        ci1 = pltpu.async_copy(dst_hbm.at[pl.ds(off1, CB)], didx1, si1)
        cm1 = pltpu.async_copy(msg_hbm.at[pl.ds(off1, CB)], mbuf1, sm1)
        ci0.wait()
        cm0.wait()
        pltpu.sync_copy(mbuf0, sh_out.at[didx0], add=True)
        ci1.wait()
        cm1.wait()
        pltpu.sync_copy(mbuf1, sh_out.at[didx1], add=True)
        return carry

    lax.fori_loop(0, CH // 2, body, 0)
    if CH % 2:
        off = base + (CH - 1) * CB
        ci = pltpu.async_copy(dst_hbm.at[pl.ds(off, CB)], didx0, si0)
        cm = pltpu.async_copy(msg_hbm.at[pl.ds(off, CB)], mbuf0, sm0)
        ci.wait()
        cm.wait()
        pltpu.sync_copy(mbuf0, sh_out.at[didx0], add=True)
    plsc.subcore_barrier()
    for i in range(NPS // CB):
        rows = pl.ds(s * NPS + i * CB, CB)
        pltpu.sync_copy(sh_out.at[rows], mbuf0)
        pltpu.sync_copy(mbuf0, u_hbm.at[pl.ds(c * NP + s * NPS + i * CB, CB)])


def _head_mask(H, DH):
    M = np.zeros((D, 16), np.float32)
    for h in range(H):
        M[h * DH:(h + 1) * DH, h] = 1.0
    return M


_M1 = _head_mask(8, 16)
_M2 = _head_mask(1, 128)


def kernel(x, edge_index, W_src1, W_dst1, attn1, bias1, ln1_g, ln1_b,
           W_src2, W_dst2, attn2, bias2, ln2_g, ln2_b):
    ei = edge_index.astype(jnp.int32)
    srcp = jnp.pad(ei[0], (0, EP - E))
    dstp = jnp.pad(ei[1], (0, EP - E))
    srcr = srcp.reshape(NW, CH, CB)
    dstr = dstp.reshape(NW, CH, CB)


    def layer(h, Wsrc, Wdst, attn, bias, g, b, res, residual, M):
        fs, fd = _project(h, Wsrc, Wdst)
        A = attn.reshape(D)[:, None] * jnp.asarray(M)
        Rm = jnp.asarray(M.T)
        rows_s, rows_d = _sc_gather(fs, fd, srcr, dstr)
        msg, ex = _edge_math(rows_s, rows_d, A, Rm)
        u = _sc_scatter_msg(msg, dstp)
        dd = _sc_scatter_msg(ex, dstp)
        u0, u1 = u[:NP], u[NP:]
        d0, d1 = dd[:NP], dd[NP:]
        return _postprocess(u0, u1, d0, d1, bias.reshape(1, D),
                            g.reshape(1, D), b.reshape(1, D), res, residual)

    h = layer(x, W_src1, W_dst1, attn1, bias1, ln1_g, ln1_b, x, True, _M1)
    out = layer(h, W_src2, W_dst2, attn2, bias2, ln2_g, ln2_b, h, False, _M2)
    return out
